# baseline (speedup 1.0000x reference)
import functools

import jax
import jax.numpy as jnp
from jax import lax
from jax.experimental import pallas as pl
from jax.experimental.pallas import tpu as pltpu
from jax.experimental.pallas import tpu_sc as plsc

N_NODES = 100000
D = 512
K = 64

NC = 1
NS = 16
NW = 8
ROWS_PER_W = K // NW


def _sc_gather_body(mem_hbm, nbr_hbm, out_hbm, idx_v, rows_v, sem):
    wid = lax.axis_index("s") * NC + lax.axis_index("c")

    @pl.when(wid < NW)
    def _():
        base = wid * ROWS_PER_W
        pltpu.sync_copy(nbr_hbm.at[pl.ds(base, ROWS_PER_W)], idx_v)
        pltpu.async_copy(mem_hbm.at[idx_v], rows_v, sem).wait()
        pltpu.sync_copy(rows_v, out_hbm.at[pl.ds(base, ROWS_PER_W)])


@functools.cache
def _sc_gather():
    return pl.kernel(
        _sc_gather_body,
        out_type=jax.ShapeDtypeStruct((K, D), jnp.float32),
        mesh=plsc.VectorSubcoreMesh(
            core_axis_name="c", subcore_axis_name="s",
            num_cores=NC, num_subcores=NS,
        ),
        scratch_types=[
            pltpu.VMEM((ROWS_PER_W,), jnp.int32),
            pltpu.VMEM((ROWS_PER_W, D), jnp.float32),
            pltpu.SemaphoreType.DMA,
        ],
    )


def _tc_combine_body(lt_ref, ct_ref, e_ref, w_ref, b_ref, o_ref):
    ct = ct_ref[0, 0]
    w = jnp.exp(-jnp.maximum(ct - lt_ref[...], 0.0))
    s = jnp.sum(w)
    norm = jnp.maximum(s, 1e-8)
    ve = lax.dot_general(
        w, e_ref[...], (((1,), (0,)), ((), ())),
        preferred_element_type=jnp.float32,
    )
    out = lax.dot_general(
        ve, w_ref[...], (((1,), (1,)), ((), ())),
        preferred_element_type=jnp.float32,
    )
    o_ref[...] = (out + s * b_ref[...]) / norm


_tc_combine = pl.pallas_call(
    _tc_combine_body,
    out_shape=jax.ShapeDtypeStruct((1, D), jnp.float32),
)


def kernel(center_idx, center_emb, memory, neighbors, last_times, current_time,
           W_msg, b_msg):
    del center_idx, center_emb
    gathered = _sc_gather()(memory, neighbors)
    ct = jnp.asarray(current_time, jnp.float32).reshape(1, 1)
    out = _tc_combine(
        last_times.reshape(1, K), ct, gathered, W_msg, b_msg.reshape(1, D)
    )
    return out.reshape(D)
